# Initial kernel scaffold; baseline (speedup 1.0000x reference)
import jax
import jax.numpy as jnp
from jax.experimental import pallas as pl


def kernel(x_drug, x_disease, x_protein, edge_drug_disease, edge_drug_protein, edge_drug_drug, edge_protein_disease, edge_protein_protein, W_in_drug, b_in_drug, W_in_disease, b_in_disease, W_in_protein, b_in_protein, W_k_0_drug, b_k_0_drug, W_q_0_drug, b_q_0_drug, W_v_0_drug, b_v_0_drug, W_a_0_drug, b_a_0_drug, skip_0_drug, W_k_0_disease, b_k_0_disease, W_q_0_disease, b_q_0_disease, W_v_0_disease, b_v_0_disease, W_a_0_disease, b_a_0_disease, skip_0_disease, W_k_0_protein, b_k_0_protein, W_q_0_protein, b_q_0_protein, W_v_0_protein, b_v_0_protein, W_a_0_protein, b_a_0_protein, skip_0_protein, a_rel_0_drug_disease, m_rel_0_drug_disease, p_rel_0_drug_disease, a_rel_0_drug_protein, m_rel_0_drug_protein, p_rel_0_drug_protein, a_rel_0_drug_drug, m_rel_0_drug_drug, p_rel_0_drug_drug, a_rel_0_protein_disease, m_rel_0_protein_disease, p_rel_0_protein_disease, a_rel_0_protein_protein, m_rel_0_protein_protein, p_rel_0_protein_protein, W_k_1_drug, b_k_1_drug, W_q_1_drug, b_q_1_drug, W_v_1_drug, b_v_1_drug, W_a_1_drug, b_a_1_drug, skip_1_drug, W_k_1_disease, b_k_1_disease, W_q_1_disease, b_q_1_disease, W_v_1_disease, b_v_1_disease, W_a_1_disease, b_a_1_disease, skip_1_disease, W_k_1_protein, b_k_1_protein, W_q_1_protein, b_q_1_protein, W_v_1_protein, b_v_1_protein, W_a_1_protein, b_a_1_protein, skip_1_protein, a_rel_1_drug_disease, m_rel_1_drug_disease, p_rel_1_drug_disease, a_rel_1_drug_protein, m_rel_1_drug_protein, p_rel_1_drug_protein, a_rel_1_drug_drug, m_rel_1_drug_drug, p_rel_1_drug_drug, a_rel_1_protein_disease, m_rel_1_protein_disease, p_rel_1_protein_disease, a_rel_1_protein_protein, m_rel_1_protein_protein, p_rel_1_protein_protein, W_out, b_out):
    raise NotImplementedError("write your pallas kernel here")


# speedup vs baseline: 11.9465x; 11.9465x over previous
import functools

import jax
import jax.numpy as jnp
import numpy as np
from jax import lax
from jax.experimental import pallas as pl
from jax.experimental.pallas import tpu as pltpu
from jax.experimental.pallas import tpu_sc as plsc

N = 10000
D = 128
H = 8
DH = 16
E = 120000
NW = 32
CH = 128
NCH = 30
EW = CH * NCH
EP = NW * EW
PADN = EP - E
NP = 10112
RPT = NP // 16
BN = 1000
GRID = N // BN

_MESH = plsc.VectorSubcoreMesh(core_axis_name="c", subcore_axis_name="s",
                               num_cores=2, num_subcores=16)
_f32 = jnp.float32
_i32 = jnp.int32


def _dense_in(x, W_in, b_in, Ws, bs):
    nproj = len(Ws)

    def body(x_ref, wi_ref, bi_ref, *rest):
        wbs = rest[:2 * nproj]
        outs = rest[2 * nproj:]
        x0 = jax.nn.relu(
            jnp.dot(x_ref[...], wi_ref[...], preferred_element_type=_f32)
            + bi_ref[...])
        outs[0][...] = x0
        for j in range(nproj):
            outs[1 + j][...] = (
                jnp.dot(x0, wbs[2 * j][...], preferred_element_type=_f32)
                + wbs[2 * j + 1][...])

    wspec = pl.BlockSpec((D, D), lambda i: (0, 0))
    bspec = pl.BlockSpec((1, D), lambda i: (0, 0))
    xspec = pl.BlockSpec((BN, D), lambda i: (i, 0))
    in_specs = [xspec, wspec, bspec] + [wspec, bspec] * nproj
    args = [x, W_in, b_in.reshape(1, D)]
    for Wj, bj in zip(Ws, bs):
        args += [Wj, bj.reshape(1, D)]
    return pl.pallas_call(
        body, grid=(GRID,), in_specs=in_specs,
        out_specs=[xspec] * (1 + nproj),
        out_shape=[jax.ShapeDtypeStruct((N, D), _f32)] * (1 + nproj),
        compiler_params=pltpu.CompilerParams(
            dimension_semantics=("arbitrary",)),
    )(*args)


def _dense_agg(n0, n1, d0, d1, xprev, Wa, ba, s1m, Ws, bs, sigmoid_out):
    nproj = len(Ws)
    out_x = not sigmoid_out

    def body(n0_ref, n1_ref, d0_ref, d1_ref, xp_ref, wa_ref, ba_ref,
             s1_ref, *rest):
        wbs = rest[:2 * nproj]
        outs = rest[2 * nproj:]
        den = d0_ref[...] + d1_ref[...]
        r = lax.broadcasted_iota(_i32, (16, D), 0)
        cg = lax.broadcasted_iota(_i32, (16, D), 1) // DH
        Bm = (r == cg).astype(_f32)
        denf = jnp.dot(den, Bm, preferred_element_type=_f32)
        num = n0_ref[...] + n1_ref[...]
        agg = jnp.where(denf > 0, num / denf, 0.0)
        g = jax.nn.gelu(agg)
        xn = (jnp.dot(g, wa_ref[...], preferred_element_type=_f32)
              + ba_ref[...] + xp_ref[...] * s1_ref[...])
        k = 0
        if out_x:
            outs[0][...] = xn
            k = 1
        for j in range(nproj):
            v = (jnp.dot(xn, wbs[2 * j][...], preferred_element_type=_f32)
                 + wbs[2 * j + 1][...])
            if sigmoid_out:
                v = jax.nn.sigmoid(v)
            outs[k + j][...] = v

    wspec = pl.BlockSpec((D, D), lambda i: (0, 0))
    bspec = pl.BlockSpec((1, D), lambda i: (0, 0))
    xspec = pl.BlockSpec((BN, D), lambda i: (i, 0))
    dspec = pl.BlockSpec((BN, 16), lambda i: (i, 0))
    in_specs = [xspec, xspec, dspec, dspec, xspec, wspec, bspec, bspec]
    in_specs += [wspec, bspec] * nproj
    args = [n0, n1, d0, d1, xprev, Wa, ba.reshape(1, D), s1m]
    for Wj, bj in zip(Ws, bs):
        args += [Wj, bj.reshape(1, D)]
    nout = (1 if out_x else 0) + nproj
    return pl.pallas_call(
        body, grid=(GRID,), in_specs=in_specs,
        out_specs=[xspec] * nout,
        out_shape=[jax.ShapeDtypeStruct((N, D), _f32)] * nout,
        compiler_params=pltpu.CompilerParams(
            dimension_semantics=("arbitrary",)),
    )(*args)


def _pass_a_body(nrels, *refs):
    ins = refs[:4 * nrels]
    outs = refs[4 * nrels:4 * nrels + nrels + 1]
    idx_s, idx_d, kbuf, qbuf, abuf, maxbuf, sem0, sem1 = refs[4 * nrels + nrels + 1:]
    mx = outs[nrels]
    c = lax.axis_index("c")
    s = lax.axis_index("s")
    wid = s * 2 + c
    i16 = lax.iota(_i32, 16)

    for rel in range(nrels):
        ktab, qtab, sref, dref = ins[4 * rel:4 * rel + 4]
        aref = outs[rel]
        for h in range(H):
            maxbuf[h] = jnp.full((16,), -1e30, _f32)

        def chunk(ci, carry):
            base = wid * EW + ci * CH
            pltpu.sync_copy(sref.at[pl.ds(base, CH)], idx_s)
            pltpu.sync_copy(dref.at[pl.ds(base, CH)], idx_d)
            ck = pltpu.async_copy(ktab.at[idx_s], kbuf, sem0)
            cq = pltpu.async_copy(qtab.at[idx_d], qbuf, sem1)
            ck.wait()
            cq.wait()

            def grp(g, carry2):
                rows = i16 + g * 16
                for h in range(H):
                    acc = jnp.zeros((16,), _f32)
                    for dsub in range(DH):
                        col = jnp.full((16,), h * DH + dsub, _i32)
                        acc = acc + (plsc.load_gather(kbuf, [rows, col])
                                     * plsc.load_gather(qbuf, [rows, col]))
                    abuf[h, pl.ds(g * 16, 16)] = acc
                    maxbuf[h] = jnp.maximum(maxbuf[h], acc)
                return carry2

            lax.fori_loop(0, CH // 16, grp, 0)
            pltpu.sync_copy(abuf, aref.at[wid, ci])
            return carry

        lax.fori_loop(0, NCH, chunk, 0)
        pltpu.sync_copy(maxbuf, mx.at[rel, wid])


def _make_pass_a(nrels):
    out_type = ([jax.ShapeDtypeStruct((NW, NCH, H, CH), _f32)] * nrels
                + [jax.ShapeDtypeStruct((nrels, NW, H, 16), _f32)])
    scratch = [
        pltpu.VMEM((CH,), _i32), pltpu.VMEM((CH,), _i32),
        pltpu.VMEM((CH, D), _f32), pltpu.VMEM((CH, D), _f32),
        pltpu.VMEM((H, CH), _f32), pltpu.VMEM((H, 16), _f32),
        pltpu.SemaphoreType.DMA, pltpu.SemaphoreType.DMA,
    ]
    return pl.kernel(functools.partial(_pass_a_body, nrels),
                     out_type=out_type, mesh=_MESH, scratch_types=scratch,
                     compiler_params=pltpu.CompilerParams(
                         needs_layout_passes=False))


def _pass_b_body(nrels, *refs):
    ins = refs[:4 * nrels + 2]
    num0, num1 = refs[4 * nrels + 2:4 * nrels + 4]
    (idx_s, idx_d, vbuf, abuf, exbuf, cv,
     num_sh, sem0, sem1) = refs[4 * nrels + 4:]
    cvec = ins[4 * nrels]
    zrows = ins[4 * nrels + 1]
    c = lax.axis_index("c")
    s = lax.axis_index("s")
    wid = s * 2 + c
    i16 = lax.iota(_i32, 16)
    z16 = jnp.zeros((16,), _f32)

    pltpu.sync_copy(zrows, vbuf)
    pltpu.sync_copy(cvec, cv)
    for j in range(5):
        rows = CH if j < 4 else RPT - 4 * CH
        off = s * RPT + j * CH
        pltpu.sync_copy(vbuf.at[pl.ds(0, rows)],
                        num_sh.at[pl.ds(off, rows)])
    for h in range(H, 16):
        for g in range(CH // 16):
            exbuf[h, pl.ds(g * 16, 16)] = z16
    plsc.subcore_barrier()

    for rel in range(nrels):
        vtab, sref, dref, aref = ins[4 * rel:4 * rel + 4]

        def chunk(ci, carry):
            base = wid * EW + ci * CH
            pltpu.sync_copy(sref.at[pl.ds(base, CH)], idx_s)
            pltpu.sync_copy(dref.at[pl.ds(base, CH)], idx_d)
            cpv = pltpu.async_copy(vtab.at[idx_s], vbuf, sem0)
            cpa = pltpu.async_copy(aref.at[wid, ci], abuf, sem1)
            cpa.wait()
            for h in range(H):
                chh = plsc.load_gather(cv, [i16, jnp.full((16,), h, _i32)])

                def eg(g, cc):
                    exbuf[h, pl.ds(g * 16, 16)] = jnp.exp(
                        abuf[h, pl.ds(g * 16, 16)] - chh)
                    return cc

                lax.fori_loop(0, CH // 16, eg, 0)
            cpv.wait()

            def edge(e, cc):
                ecol = jnp.full((16,), e, _i32)
                for h in range(H):
                    w = plsc.load_gather(exbuf, [jnp.full((16,), h, _i32), ecol])
                    cols = i16 + h * DH
                    v16 = plsc.load_gather(vbuf, [ecol, cols])
                    plsc.store_scatter(vbuf, [ecol, cols], v16 * w)
                return cc

            lax.fori_loop(0, CH, edge, 0)
            pltpu.sync_copy(vbuf, num_sh.at[idx_d], add=True)
            return carry

        lax.fori_loop(0, NCH, chunk, 0)

    plsc.subcore_barrier()

    def _dump(nout):
        for j in range(5):
            rows = CH if j < 4 else RPT - 4 * CH
            off = s * RPT + j * CH
            pltpu.sync_copy(num_sh.at[pl.ds(off, rows)],
                            vbuf.at[pl.ds(0, rows)])
            pltpu.sync_copy(vbuf.at[pl.ds(0, rows)],
                            nout.at[pl.ds(off, rows)])

    @pl.when(c == 0)
    def _():
        _dump(num0)

    @pl.when(c == 1)
    def _():
        _dump(num1)


def _make_pass_b(nrels):
    out_type = [jax.ShapeDtypeStruct((NP, D), _f32),
                jax.ShapeDtypeStruct((NP, D), _f32)]
    scratch = [
        pltpu.VMEM((CH,), _i32), pltpu.VMEM((CH,), _i32),
        pltpu.VMEM((CH, D), _f32), pltpu.VMEM((H, CH), _f32),
        pltpu.VMEM((16, CH), _f32), pltpu.VMEM((16, 16), _f32),
        pltpu.VMEM_SHARED((NP, D), _f32),
        pltpu.SemaphoreType.DMA, pltpu.SemaphoreType.DMA,
    ]
    return pl.kernel(functools.partial(_pass_b_body, nrels),
                     out_type=out_type, mesh=_MESH, scratch_types=scratch,
                     compiler_params=pltpu.CompilerParams(
                         needs_layout_passes=False))


def _pass_c_body(nrels, *refs):
    ins = refs[:2 * nrels + 2]
    den0, den1 = refs[2 * nrels + 2:2 * nrels + 4]
    (idx_d, exd, abuf, cv, den_sh, sem1) = refs[2 * nrels + 4:]
    cvec = ins[2 * nrels]
    zrows = ins[2 * nrels + 1]
    c = lax.axis_index("c")
    s = lax.axis_index("s")
    wid = s * 2 + c
    i16 = lax.iota(_i32, 16)

    pltpu.sync_copy(zrows, exd)
    pltpu.sync_copy(cvec, cv)
    for j in range(5):
        rows = CH if j < 4 else RPT - 4 * CH
        off = s * RPT + j * CH
        pltpu.sync_copy(exd.at[pl.ds(0, rows)],
                        den_sh.at[pl.ds(off, rows)])
    plsc.subcore_barrier()

    for rel in range(nrels):
        dref, aref = ins[2 * rel], ins[2 * rel + 1]

        def chunk(ci, carry):
            base = wid * EW + ci * CH
            pltpu.sync_copy(dref.at[pl.ds(base, CH)], idx_d)
            pltpu.async_copy(aref.at[wid, ci], abuf, sem1).wait()
            for h in range(H):
                chh = plsc.load_gather(cv, [i16, jnp.full((16,), h, _i32)])
                hc = jnp.full((16,), h, _i32)

                def eg(g, cc):
                    ex = jnp.exp(abuf[h, pl.ds(g * 16, 16)] - chh)
                    plsc.store_scatter(exd, [i16 + g * 16, hc], ex)
                    return cc

                lax.fori_loop(0, CH // 16, eg, 0)
            pltpu.sync_copy(exd, den_sh.at[idx_d], add=True)
            return carry

        lax.fori_loop(0, NCH, chunk, 0)

    plsc.subcore_barrier()

    def _dump(dout):
        for j in range(5):
            rows = CH if j < 4 else RPT - 4 * CH
            off = s * RPT + j * CH
            pltpu.sync_copy(den_sh.at[pl.ds(off, rows)],
                            exd.at[pl.ds(0, rows)])
            pltpu.sync_copy(exd.at[pl.ds(0, rows)],
                            dout.at[pl.ds(off, rows)])

    @pl.when(c == 0)
    def _():
        _dump(den0)

    @pl.when(c == 1)
    def _():
        _dump(den1)


def _make_pass_c(nrels):
    out_type = [jax.ShapeDtypeStruct((NP, D), _f32),
                jax.ShapeDtypeStruct((NP, D), _f32)]
    scratch = [
        pltpu.VMEM((CH,), _i32), pltpu.VMEM((CH, D), _f32),
        pltpu.VMEM((H, CH), _f32), pltpu.VMEM((16, 16), _f32),
        pltpu.VMEM_SHARED((NP, D), _f32),
        pltpu.SemaphoreType.DMA,
    ]
    return pl.kernel(functools.partial(_pass_c_body, nrels),
                     out_type=out_type, mesh=_MESH, scratch_types=scratch,
                     compiler_params=pltpu.CompilerParams(
                         needs_layout_passes=False))


_PASS_A = _make_pass_a(3)
_PASS_B1 = _make_pass_b(1)
_PASS_B2 = _make_pass_b(2)
_PASS_C1 = _make_pass_c(1)
_PASS_C2 = _make_pass_c(2)


def _bd(a):
    return jax.scipy.linalg.block_diag(*[a[h] for h in range(H)])


def _pad_edges(ei):
    src = jnp.concatenate([ei[0], jnp.zeros((PADN,), _i32)])
    dstA = jnp.concatenate([ei[1], jnp.zeros((PADN,), _i32)])
    dstB = jnp.concatenate([ei[1], jnp.full((PADN,), N, _i32)])
    return src, dstA, dstB


def kernel(x_drug, x_disease, x_protein, edge_drug_disease, edge_drug_protein, edge_drug_drug, edge_protein_disease, edge_protein_protein, W_in_drug, b_in_drug, W_in_disease, b_in_disease, W_in_protein, b_in_protein, W_k_0_drug, b_k_0_drug, W_q_0_drug, b_q_0_drug, W_v_0_drug, b_v_0_drug, W_a_0_drug, b_a_0_drug, skip_0_drug, W_k_0_disease, b_k_0_disease, W_q_0_disease, b_q_0_disease, W_v_0_disease, b_v_0_disease, W_a_0_disease, b_a_0_disease, skip_0_disease, W_k_0_protein, b_k_0_protein, W_q_0_protein, b_q_0_protein, W_v_0_protein, b_v_0_protein, W_a_0_protein, b_a_0_protein, skip_0_protein, a_rel_0_drug_disease, m_rel_0_drug_disease, p_rel_0_drug_disease, a_rel_0_drug_protein, m_rel_0_drug_protein, p_rel_0_drug_protein, a_rel_0_drug_drug, m_rel_0_drug_drug, p_rel_0_drug_drug, a_rel_0_protein_disease, m_rel_0_protein_disease, p_rel_0_protein_disease, a_rel_0_protein_protein, m_rel_0_protein_protein, p_rel_0_protein_protein, W_k_1_drug, b_k_1_drug, W_q_1_drug, b_q_1_drug, W_v_1_drug, b_v_1_drug, W_a_1_drug, b_a_1_drug, skip_1_drug, W_k_1_disease, b_k_1_disease, W_q_1_disease, b_q_1_disease, W_v_1_disease, b_v_1_disease, W_a_1_disease, b_a_1_disease, skip_1_disease, W_k_1_protein, b_k_1_protein, W_q_1_protein, b_q_1_protein, W_v_1_protein, b_v_1_protein, W_a_1_protein, b_a_1_protein, skip_1_protein, a_rel_1_drug_disease, m_rel_1_drug_disease, p_rel_1_drug_disease, a_rel_1_drug_protein, m_rel_1_drug_protein, p_rel_1_drug_protein, a_rel_1_drug_drug, m_rel_1_drug_drug, p_rel_1_drug_drug, a_rel_1_protein_disease, m_rel_1_protein_disease, p_rel_1_protein_disease, a_rel_1_protein_protein, m_rel_1_protein_protein, p_rel_1_protein_protein, W_out, b_out):
    aw = {0: (a_rel_0_drug_drug, p_rel_0_drug_drug, m_rel_0_drug_drug,
              a_rel_0_drug_protein, p_rel_0_drug_protein, m_rel_0_drug_protein,
              a_rel_0_protein_protein, p_rel_0_protein_protein,
              m_rel_0_protein_protein),
          1: (a_rel_1_drug_drug, p_rel_1_drug_drug, m_rel_1_drug_drug,
              a_rel_1_drug_protein, p_rel_1_drug_protein, m_rel_1_drug_protein,
              a_rel_1_protein_protein, p_rel_1_protein_protein,
              m_rel_1_protein_protein)}
    kv = {0: (W_k_0_drug, b_k_0_drug, W_v_0_drug, b_v_0_drug,
              W_q_0_drug, b_q_0_drug,
              W_k_0_protein, b_k_0_protein, W_v_0_protein, b_v_0_protein,
              W_q_0_protein, b_q_0_protein),
          1: (W_k_1_drug, b_k_1_drug, W_v_1_drug, b_v_1_drug,
              W_q_1_drug, b_q_1_drug,
              W_k_1_protein, b_k_1_protein, W_v_1_protein, b_v_1_protein,
              W_q_1_protein, b_q_1_protein)}

    def fold(l):
        (add_, pdd, mdd, adp, pdp, mdp, app_, ppp, mpp) = aw[l]
        (Wkd, bkd, Wvd, bvd, Wqd, bqd, Wkp, bkp, Wvp, bvp, Wqp, bqp) = kv[l]
        inv = 1.0 / np.sqrt(DH)
        Add = _bd(add_ * (pdd[:, None, None] * inv))
        Adp = _bd(adp * (pdp[:, None, None] * inv))
        App = _bd(app_ * (ppp[:, None, None] * inv))
        Mdd, Mdp, Mpp = _bd(mdd), _bd(mdp), _bd(mpp)
        Wd = [Wqd, Wkd @ Add, Wvd @ Mdd, Wkd @ Adp, Wvd @ Mdp]
        bd_ = [bqd, bkd @ Add, bvd @ Mdd, bkd @ Adp, bvd @ Mdp]
        Wp = [Wqp, Wkp @ App, Wvp @ Mpp]
        bp = [bqp, bkp @ App, bvp @ Mpp]
        return Wd, bd_, Wp, bp

    s_dd, dA_dd, dB_dd = _pad_edges(edge_drug_drug)
    s_dp, dA_dp, dB_dp = _pad_edges(edge_drug_protein)
    s_pp, dA_pp, dB_pp = _pad_edges(edge_protein_protein)
    zrows = jnp.zeros((CH, D), _f32)

    Wd0, bd0, Wp0, bp0 = fold(0)
    x_d, q_d, kdd, vdd, kdp, vdp = _dense_in(
        x_drug, W_in_drug, b_in_drug, Wd0, bd0)
    x_p, q_p, kpp, vpp = _dense_in(
        x_protein, W_in_protein, b_in_protein, Wp0, bp0)

    def edge_layer(q_d, q_p, kdd, vdd, kdp, vdp, kpp, vpp):
        a_dd, a_dp, a_pp, mx = _PASS_A(
            kdd, q_d, s_dd, dA_dd,
            kdp, q_p, s_dp, dA_dp,
            kpp, q_p, s_pp, dA_pp)
        Cd = jnp.pad(mx[0].max(axis=(0, 2)), (0, 8))
        Cp = jnp.pad(jnp.maximum(mx[1], mx[2]).max(axis=(0, 2)), (0, 8))
        CdM = jnp.tile(Cd[None, :], (16, 1))
        CpM = jnp.tile(Cp[None, :], (16, 1))
        nd0, nd1 = _PASS_B1(vdd, s_dd, dB_dd, a_dd, CdM, zrows)
        np0, np1 = _PASS_B2(vdp, s_dp, dB_dp, a_dp,
                            vpp, s_pp, dB_pp, a_pp, CpM, zrows)
        dd0, dd1 = _PASS_C1(dB_dd, a_dd, CdM, zrows)
        dp0, dp1 = _PASS_C2(dB_dp, a_dp, dB_pp, a_pp, CpM, zrows)
        return ((nd0, nd1, dd0[:, :16], dd1[:, :16]),
                (np0, np1, dp0[:, :16], dp1[:, :16]))

    aggd, aggp = edge_layer(q_d, q_p, kdd, vdd, kdp, vdp, kpp, vpp)

    Wd1, bd1, Wp1, bp1 = fold(1)
    beta_d0 = jax.nn.sigmoid(skip_0_drug)
    beta_p0 = jax.nn.sigmoid(skip_0_protein)
    x_d1, q_d1, kdd1, vdd1, kdp1, vdp1 = _dense_agg(
        aggd[0], aggd[1], aggd[2], aggd[3], x_d,
        beta_d0 * W_a_0_drug, beta_d0 * b_a_0_drug,
        jnp.full((1, D), 1.0 - beta_d0, _f32), Wd1, bd1, sigmoid_out=False)
    x_p1, q_p1, kpp1, vpp1 = _dense_agg(
        aggp[0], aggp[1], aggp[2], aggp[3], x_p,
        beta_p0 * W_a_0_protein, beta_p0 * b_a_0_protein,
        jnp.full((1, D), 1.0 - beta_p0, _f32), Wp1, bp1, sigmoid_out=False)

    aggd1, aggp1 = edge_layer(q_d1, q_p1, kdd1, vdd1, kdp1, vdp1, kpp1, vpp1)

    Wout_pad = jnp.pad(W_out, ((0, 0), (0, D - W_out.shape[1])))
    bout_pad = jnp.pad(b_out, (0, D - b_out.shape[0]))
    beta_d1 = jax.nn.sigmoid(skip_1_drug)
    beta_p1 = jax.nn.sigmoid(skip_1_protein)
    (o_d,) = _dense_agg(
        aggd1[0], aggd1[1], aggd1[2], aggd1[3], x_d1,
        beta_d1 * W_a_1_drug, beta_d1 * b_a_1_drug,
        jnp.full((1, D), 1.0 - beta_d1, _f32), [Wout_pad], [bout_pad],
        sigmoid_out=True)
    (o_p,) = _dense_agg(
        aggp1[0], aggp1[1], aggp1[2], aggp1[3], x_p1,
        beta_p1 * W_a_1_protein, beta_p1 * b_a_1_protein,
        jnp.full((1, D), 1.0 - beta_p1, _f32), [Wout_pad], [bout_pad],
        sigmoid_out=True)
    return (o_d[:, :2], o_p[:, :2])
